# baseline (speedup 1.0000x reference)
import functools

import jax
import jax.numpy as jnp
from jax import lax
from jax.experimental import pallas as pl
from jax.experimental.pallas import tpu as pltpu
from jax.experimental.pallas import tpu_sc as plsc

N_NODES = 100000
D = 128
S = 512
TILE = 128
NUM_TILES = N_NODES // TILE
TAIL = N_NODES - NUM_TILES * TILE
NC = 2
NS = 16
NW = NC * NS
ROWS_PER_SUBCORE = S // NS


def _sc_segment_partials(x, batch):
    mesh = plsc.VectorSubcoreMesh(core_axis_name="c", subcore_axis_name="s")

    base_tiles = NUM_TILES // NW
    rem_tiles = NUM_TILES - base_tiles * NW
    max_tiles = base_tiles + 1
    NBUF = 6
    LOOK = 3

    @functools.partial(
        pl.kernel,
        out_type=jax.ShapeDtypeStruct((NC, S, D), jnp.float32),
        mesh=mesh,
        scratch_types=[
            pltpu.VMEM((NBUF, TILE), jnp.int32),
            pltpu.VMEM((NBUF, TILE, D), jnp.float32),
            pltpu.VMEM((ROWS_PER_SUBCORE, D), jnp.float32),
            pltpu.VMEM_SHARED((S, D), jnp.float32),
            pltpu.SemaphoreType.DMA((NBUF,)),
            pltpu.SemaphoreType.DMA((NBUF,)),
            pltpu.SemaphoreType.DMA((NBUF,)),
        ],
    )
    def k(x_hbm, b_hbm, out_hbm, idx_v, rows_v, zb_v, acc_sh, lsems, isems, ssems):
        c = lax.axis_index("c")
        s = lax.axis_index("s")
        wid = s * NC + c
        start = wid * base_tiles + jnp.minimum(wid, rem_tiles)
        cnt = jnp.where(wid < rem_tiles, base_tiles + 1, base_tiles)

        def issue_loads(j):
            b = j % NBUF
            pltpu.async_copy(x_hbm.at[pl.ds((start + j) * TILE, TILE)],
                             rows_v.at[b], lsems.at[b])
            pltpu.async_copy(b_hbm.at[pl.ds((start + j) * TILE, TILE)],
                             idx_v.at[b], isems.at[b])

        def wait_rows(sem):
            pltpu.make_async_copy(x_hbm.at[pl.ds(0, TILE)], rows_v.at[0],
                                  sem).wait()

        def wait_ids(sem):
            pltpu.make_async_copy(b_hbm.at[pl.ds(0, TILE)], idx_v.at[0],
                                  sem).wait()

        with jax.named_scope("prime"):
            for j in range(min(LOOK, max_tiles)):
                @pl.when(j < cnt)
                def _(j=j):
                    issue_loads(j)

        with jax.named_scope("zero"):
            @pl.loop(0, ROWS_PER_SUBCORE)
            def _(r):
                for v in range(D // 16):
                    zb_v[r, pl.ds(v * 16, 16)] = jnp.zeros((16,), jnp.float32)

            pltpu.sync_copy(zb_v, acc_sh.at[pl.ds(s * ROWS_PER_SUBCORE, ROWS_PER_SUBCORE)])
        with jax.named_scope("barrier1"):
            plsc.subcore_barrier()

        scope_main = jax.named_scope("main")
        scope_main.__enter__()

        @pl.loop(0, base_tiles // NBUF)
        def _(it):
            jbase = it * NBUF
            for u in range(NBUF):
                j = jbase + u
                wait_rows(lsems.at[u])
                wait_ids(isems.at[u])
                pltpu.async_copy(rows_v.at[u], acc_sh.at[idx_v.at[u]],
                                 ssems.at[u], add=True)

                t = j + LOOK
                tb = (u + LOOK) % NBUF

                def refill(t=t, tb=tb, guard_prev=True):
                    if guard_prev:
                        wait_rows(ssems.at[tb])
                    issue_loads(t)

                if u < NBUF - LOOK:
                    @pl.when(it > 0)
                    def _(t=t, tb=tb):
                        refill(t, tb, True)

                    @pl.when(it == 0)
                    def _(t=t, tb=tb):
                        refill(t, tb, False)
                else:
                    @pl.when(t < cnt)
                    def _(t=t, tb=tb):
                        refill(t, tb, True)

        @pl.when(cnt > base_tiles)
        def _():
            b = base_tiles % NBUF
            wait_rows(lsems.at[b])
            wait_ids(isems.at[b])
            pltpu.async_copy(rows_v.at[b], acc_sh.at[idx_v.at[b]],
                             ssems.at[b], add=True)

        scope_main.__exit__(None, None, None)

        with jax.named_scope("drain"):
            for u in range(NBUF):
                wait_rows(ssems.at[u])

        with jax.named_scope("barrier2"):
            plsc.subcore_barrier()

        with jax.named_scope("readout"):
            sl = pl.ds(s * ROWS_PER_SUBCORE, ROWS_PER_SUBCORE)
            pltpu.sync_copy(acc_sh.at[sl], out_hbm.at[c, sl])

    return k(x, batch)


def _tc_finish(parts, tail_x, tail_ids, W, b):
    def body(p_ref, tx_ref, ti_ref, w_ref, b_ref, o_ref):
        ids = ti_ref[...]
        iota = lax.broadcasted_iota(jnp.int32, (S, TAIL), 0)
        onehot = (iota == ids).astype(jnp.float32)
        pooled = p_ref[0] + p_ref[1]
        pooled = pooled + lax.dot_general(
            onehot, tx_ref[...], (((1,), (0,)), ((), ())),
            preferred_element_type=jnp.float32)
        o_ref[...] = lax.dot_general(
            pooled, w_ref[...], (((1,), (1,)), ((), ())),
            preferred_element_type=jnp.float32) + b_ref[...]

    return pl.pallas_call(
        body,
        out_shape=jax.ShapeDtypeStruct((S, D), jnp.float32),
    )(parts, tail_x, tail_ids, W, b)


def kernel(x, batch, W, b):
    batch = batch.astype(jnp.int32)
    parts = _sc_segment_partials(x, batch)
    tail_x = x[NUM_TILES * TILE:]
    tail_ids = batch[NUM_TILES * TILE:].reshape(1, TAIL)
    return _tc_finish(parts, tail_x, tail_ids, W, b.reshape(1, D))


# speedup vs baseline: 1.0033x; 1.0033x over previous
import functools

import jax
import jax.numpy as jnp
from jax import lax
from jax.experimental import pallas as pl
from jax.experimental.pallas import tpu as pltpu
from jax.experimental.pallas import tpu_sc as plsc

N_NODES = 100000
D = 128
S = 512
TILE = 128
NUM_TILES = N_NODES // TILE
TAIL = N_NODES - NUM_TILES * TILE
NC = 2
NS = 16
NW = NC * NS
ROWS_PER_SUBCORE = S // NS


def _sc_segment_partials(x, batch):
    mesh = plsc.VectorSubcoreMesh(core_axis_name="c", subcore_axis_name="s")

    base_tiles = NUM_TILES // NW
    rem_tiles = NUM_TILES - base_tiles * NW
    max_tiles = base_tiles + 1
    NBUF = 6
    LOOK = 3

    @functools.partial(
        pl.kernel,
        out_type=jax.ShapeDtypeStruct((NC, S, D), jnp.float32),
        mesh=mesh,
        scratch_types=[
            pltpu.VMEM((NBUF, TILE), jnp.int32),
            pltpu.VMEM((NBUF, TILE, D), jnp.float32),
            pltpu.VMEM((ROWS_PER_SUBCORE, D), jnp.float32),
            pltpu.VMEM_SHARED((S, D), jnp.float32),
            pltpu.SemaphoreType.DMA((NBUF,)),
            pltpu.SemaphoreType.DMA((NBUF,)),
            pltpu.SemaphoreType.DMA((NBUF,)),
        ],
    )
    def k(x_hbm, b_hbm, out_hbm, idx_v, rows_v, zb_v, acc_sh, lsems, isems, ssems):
        c = lax.axis_index("c")
        s = lax.axis_index("s")
        wid = s * NC + c
        start = wid * base_tiles + jnp.minimum(wid, rem_tiles)
        cnt = jnp.where(wid < rem_tiles, base_tiles + 1, base_tiles)

        def issue_loads(j):
            b = j % NBUF
            pltpu.async_copy(x_hbm.at[pl.ds((start + j) * TILE, TILE)],
                             rows_v.at[b], lsems.at[b])
            pltpu.async_copy(b_hbm.at[pl.ds((start + j) * TILE, TILE)],
                             idx_v.at[b], isems.at[b])

        def wait_rows(sem):
            pltpu.make_async_copy(x_hbm.at[pl.ds(0, TILE)], rows_v.at[0],
                                  sem).wait()

        def wait_ids(sem):
            pltpu.make_async_copy(b_hbm.at[pl.ds(0, TILE)], idx_v.at[0],
                                  sem).wait()

        for j in range(min(LOOK, max_tiles)):
            @pl.when(j < cnt)
            def _(j=j):
                issue_loads(j)

        @pl.loop(0, ROWS_PER_SUBCORE)
        def _(r):
            for v in range(D // 16):
                zb_v[r, pl.ds(v * 16, 16)] = jnp.zeros((16,), jnp.float32)

        pltpu.sync_copy(zb_v, acc_sh.at[pl.ds(s * ROWS_PER_SUBCORE, ROWS_PER_SUBCORE)])
        plsc.subcore_barrier()

        @pl.loop(0, base_tiles // NBUF)
        def _(it):
            jbase = it * NBUF
            for u in range(NBUF):
                j = jbase + u
                wait_rows(lsems.at[u])
                wait_ids(isems.at[u])
                pltpu.async_copy(rows_v.at[u], acc_sh.at[idx_v.at[u]],
                                 ssems.at[u], add=True)

                t = j + LOOK
                tb = (u + LOOK) % NBUF

                def refill(t=t, tb=tb, guard_prev=True):
                    if guard_prev:
                        wait_rows(ssems.at[tb])
                    issue_loads(t)

                if u < NBUF - LOOK:
                    @pl.when(it > 0)
                    def _(t=t, tb=tb):
                        refill(t, tb, True)

                    @pl.when(it == 0)
                    def _(t=t, tb=tb):
                        refill(t, tb, False)
                else:
                    @pl.when(t < cnt)
                    def _(t=t, tb=tb):
                        refill(t, tb, True)

        @pl.when(cnt > base_tiles)
        def _():
            b = base_tiles % NBUF
            wait_rows(lsems.at[b])
            wait_ids(isems.at[b])
            pltpu.async_copy(rows_v.at[b], acc_sh.at[idx_v.at[b]],
                             ssems.at[b], add=True)

        for u in range(NBUF):
            wait_rows(ssems.at[u])

        plsc.subcore_barrier()

        sl = pl.ds(s * ROWS_PER_SUBCORE, ROWS_PER_SUBCORE)
        pltpu.sync_copy(acc_sh.at[sl], out_hbm.at[c, sl])

    return k(x, batch)


def _tc_finish(parts, tail_x, tail_ids, W, b):
    def body(p_ref, tx_ref, ti_ref, w_ref, b_ref, o_ref):
        ids = ti_ref[...]
        iota = lax.broadcasted_iota(jnp.int32, (S, TAIL), 0)
        onehot = (iota == ids).astype(jnp.float32)
        pooled = p_ref[0] + p_ref[1]
        pooled = pooled + lax.dot_general(
            onehot, tx_ref[...], (((1,), (0,)), ((), ())),
            preferred_element_type=jnp.float32)
        o_ref[...] = lax.dot_general(
            pooled, w_ref[...], (((1,), (1,)), ((), ())),
            preferred_element_type=jnp.float32) + b_ref[...]

    return pl.pallas_call(
        body,
        out_shape=jax.ShapeDtypeStruct((S, D), jnp.float32),
    )(parts, tail_x, tail_ids, W, b)


def kernel(x, batch, W, b):
    batch = batch.astype(jnp.int32)
    parts = _sc_segment_partials(x, batch)
    tail_x = x[NUM_TILES * TILE:]
    tail_ids = batch[NUM_TILES * TILE:].reshape(1, TAIL)
    return _tc_finish(parts, tail_x, tail_ids, W, b.reshape(1, D))
